# baseline (speedup 1.0000x reference)
import functools
import jax
import jax.numpy as jnp
from jax import lax
from jax.experimental import pallas as pl
from jax.experimental.pallas import tpu as pltpu
from jax.experimental.pallas import tpu_sc as plsc

NC = 2
NS = 16
NW = NC * NS
CHUNK = 128
RB = 1000
RC = 80
CG = 5


def _mesh():
    return plsc.VectorSubcoreMesh(core_axis_name="c", subcore_axis_name="s")


@functools.lru_cache(maxsize=None)
def _make_deg_kernel(N, EC):
    base, rem = EC // NW, EC % NW
    nrc = N // RC
    rc_base, rc_rem = nrc // NS, nrc % NS

    @functools.partial(
        pl.kernel,
        out_type=(jax.ShapeDtypeStruct((N,), jnp.float32),
                  jax.ShapeDtypeStruct((N,), jnp.float32)),
        mesh=_mesh(),
        scratch_types=[
            pltpu.VMEM((1, CHUNK), jnp.int32),
            pltpu.VMEM((CHUNK,), jnp.float32),
            pltpu.VMEM((RC,), jnp.float32),
            pltpu.VMEM_SHARED((N,), jnp.float32),
        ],
    )
    def deg_kernel(dst_hbm, ones_hbm, zeros_hbm, out0_hbm, out1_hbm,
                   idx_v, ones_v, buf_v, acc_sh):
        c = lax.axis_index("c")
        s = lax.axis_index("s")
        w = s * NC + c
        pltpu.sync_copy(zeros_hbm, buf_v)
        pltpu.sync_copy(ones_hbm, ones_v)
        nrc_mine = rc_base + jnp.where(s < rc_rem, 1, 0)

        def zbody(k, carry):
            pltpu.sync_copy(buf_v, acc_sh.at[pl.ds((s + NS * k) * RC, RC)])
            return carry

        lax.fori_loop(0, nrc_mine, zbody, 0)
        plsc.subcore_barrier()
        start = base * w + jnp.minimum(w, rem)
        nch = base + jnp.where(w < rem, 1, 0)

        def body(j, carry):
            pltpu.sync_copy(dst_hbm.at[start + j], idx_v)
            pltpu.sync_copy(ones_v, acc_sh.at[idx_v.at[0]], add=True)
            return carry

        lax.fori_loop(0, nch, body, 0)
        plsc.subcore_barrier()

        def obody(k, carry):
            t = (s + NS * k) * RC
            pltpu.sync_copy(acc_sh.at[pl.ds(t, RC)], buf_v)

            @pl.when(c == 0)
            def _():
                pltpu.sync_copy(buf_v, out0_hbm.at[pl.ds(t, RC)])

            @pl.when(c == 1)
            def _():
                pltpu.sync_copy(buf_v, out1_hbm.at[pl.ds(t, RC)])

            return carry

        lax.fori_loop(0, nrc_mine, obody, 0)

    return deg_kernel


@functools.lru_cache(maxsize=None)
def _make_edge_kernel(N, D, K):
    nrc = N // RC
    rc_base, rc_rem = nrc // NS, nrc % NS
    NG = K // CG
    G2 = CG // 2

    @functools.partial(
        pl.kernel,
        out_type=jax.ShapeDtypeStruct((NC, N, D), jnp.float32),
        mesh=_mesh(),
        scratch_types=[
            pltpu.VMEM((CG, 2, CHUNK), jnp.int32),
            pltpu.VMEM((CHUNK, D), jnp.float32),
            pltpu.VMEM((CHUNK, D), jnp.float32),
            pltpu.VMEM((CHUNK, D), jnp.float32),
            pltpu.VMEM_SHARED((N, D), jnp.float32),
            pltpu.SemaphoreType.DMA,
            pltpu.SemaphoreType.DMA,
            pltpu.SemaphoreType.DMA,
            pltpu.SemaphoreType.DMA,
        ],
    )
    def edge_kernel(hwp_hbm, eidx_hbm, zeros_hbm, out_hbm,
                    idx_v, rows_a, rows_b, rows_c, acc_sh, ga, gb, sa, sb):
        c = lax.axis_index("c")
        s = lax.axis_index("s")
        w = s * NC + c
        pltpu.sync_copy(zeros_hbm, rows_a.at[pl.ds(0, RC)])
        nrc_mine = rc_base + jnp.where(s < rc_rem, 1, 0)

        def zbody(k, carry):
            pltpu.sync_copy(rows_a.at[pl.ds(0, RC)],
                            acc_sh.at[pl.ds((s + NS * k) * RC, RC)])
            return carry

        lax.fori_loop(0, nrc_mine, zbody, 0)
        plsc.subcore_barrier()

        def gather(j, rows, sem):
            return pltpu.async_copy(hwp_hbm.at[idx_v.at[j, 0]], rows, sem)

        def scat(j, rows, sem):
            return pltpu.async_copy(rows, acc_sh.at[idx_v.at[j, 1]], sem,
                                    add=True)

        rows = [rows_a, rows_b, rows_c]
        gsem = [ga, gb]
        ssem = [sa, sb]

        def group(g, carry):
            pltpu.sync_copy(eidx_hbm.at[w, pl.ds(g * CG, CG)], idx_v)
            gd = [None] * CG
            sd = [None] * CG
            gd[0] = gather(0, rows[0], gsem[0])
            for k in range(CG):
                gd[k].wait()
                if k >= 2:
                    sd[k - 2].wait()
                sd[k] = scat(k, rows[k % 3], ssem[k % 2])
                if k + 1 < CG:
                    gd[k + 1] = gather(k + 1, rows[(k + 1) % 3],
                                       gsem[(k + 1) % 2])
            sd[CG - 2].wait()
            sd[CG - 1].wait()
            return carry

        lax.fori_loop(0, NG, group, 0)
        plsc.subcore_barrier()

        def obody(k, carry):
            t = (s + NS * k) * RC
            pltpu.sync_copy(acc_sh.at[pl.ds(t, RC)], rows_a.at[pl.ds(0, RC)])
            pltpu.sync_copy(rows_a.at[pl.ds(0, RC)],
                            out_hbm.at[c, pl.ds(t, RC)])
            return carry

        lax.fori_loop(0, nrc_mine, obody, 0)

    return edge_kernel


def _t1_body(h_ref, w_ref, deg0_ref, deg1_ref, hwp_ref, dis_ref):
    d = deg0_ref[...] + deg1_ref[...] + 1.0
    dis = lax.rsqrt(d)
    dis_ref[...] = dis
    hwp_ref[...] = (
        jnp.dot(h_ref[...], w_ref[...], preferred_element_type=jnp.float32)
        * dis
    )


def _post(h, hwp, accp0, accp1, dis, b, g, be):
    out = (accp0 + accp1 + hwp) * dis + b
    mu = jnp.mean(out, axis=-1, keepdims=True)
    xm = out - mu
    var = jnp.mean(xm * xm, axis=-1, keepdims=True)
    out = xm * lax.rsqrt(var + 1e-5) * g + be
    return h + jnp.maximum(out, 0.0)


def _t2_body(h_ref, hwp_ref, accp_ref, dis_ref, b_ref, g_ref, be_ref, wn_ref,
             hn_ref, hwpn_ref):
    hn = _post(h_ref[...], hwp_ref[...], accp_ref[0], accp_ref[1],
               dis_ref[...], b_ref[...], g_ref[...], be_ref[...])
    hn_ref[...] = hn
    hwpn_ref[...] = (
        jnp.dot(hn, wn_ref[...], preferred_element_type=jnp.float32)
        * dis_ref[...]
    )


def _t3_body(h_ref, hwp_ref, accp_ref, dis_ref, b_ref, g_ref, be_ref, hn_ref):
    hn_ref[...] = _post(h_ref[...], hwp_ref[...], accp_ref[0], accp_ref[1],
                        dis_ref[...], b_ref[...], g_ref[...], be_ref[...])


def _row_spec(D):
    return pl.BlockSpec((RB, D), lambda i: (i, 0))


def kernel(x, edge_index, W0, b0, g0, be0, W1, b1, g1, be1, W2, b2, g2, be2):
    N, D = x.shape
    E = edge_index.shape[1]
    EC = E // CHUNK
    K = -(-EC // NW)
    K = -(-K // CG) * CG
    ECp = K * NW
    pad = ECp * CHUNK - E
    pad_src = N + (jnp.arange(pad, dtype=jnp.int32) % 8)
    pad_dst = jnp.arange(pad, dtype=jnp.int32) % N
    src_p = jnp.concatenate([edge_index[0], pad_src])
    dst_p = jnp.concatenate([edge_index[1], pad_dst])
    eidx = jnp.stack([src_p.reshape(ECp, CHUNK), dst_p.reshape(ECp, CHUNK)],
                     axis=1).reshape(NW, K, 2, CHUNK)
    zrow8 = jnp.zeros((8, D), jnp.float32)
    dst3d = edge_index[1].reshape(EC, 1, CHUNK)
    ones1 = jnp.ones((CHUNK,), jnp.float32)
    zdeg = jnp.zeros((RC,), jnp.float32)
    zrow = jnp.zeros((RC, D), jnp.float32)
    grid = (N // RB,)

    deg0, deg1 = _make_deg_kernel(N, EC)(dst3d, ones1, zdeg)
    deg0 = deg0.reshape(N, 1)
    deg1 = deg1.reshape(N, 1)

    full = lambda a, b: pl.BlockSpec((a, b), lambda i: (0, 0))
    dis_spec = pl.BlockSpec((RB, 1), lambda i: (i, 0))
    accp_spec = pl.BlockSpec((NC, RB, D), lambda i: (0, i, 0))

    hwp, dis = pl.pallas_call(
        _t1_body,
        grid=grid,
        in_specs=[_row_spec(D), full(D, D), dis_spec, dis_spec],
        out_specs=[_row_spec(D), dis_spec],
        out_shape=[jax.ShapeDtypeStruct((N, D), jnp.float32),
                   jax.ShapeDtypeStruct((N, 1), jnp.float32)],
    )(x, W0, deg0, deg1)

    edge_k = _make_edge_kernel(N, D, K)
    h = x
    layer_params = [(b0, g0, be0, W1), (b1, g1, be1, W2), (b2, g2, be2, None)]
    for (b, g, be, Wn) in layer_params:
        accp = edge_k(jnp.concatenate([hwp, zrow8]), eidx, zrow)
        b_2d, g_2d, be_2d = b.reshape(1, D), g.reshape(1, D), be.reshape(1, D)
        if Wn is None:
            h = pl.pallas_call(
                _t3_body,
                grid=grid,
                in_specs=[_row_spec(D), _row_spec(D), accp_spec, dis_spec,
                          full(1, D), full(1, D), full(1, D)],
                out_specs=_row_spec(D),
                out_shape=jax.ShapeDtypeStruct((N, D), jnp.float32),
            )(h, hwp, accp, dis, b_2d, g_2d, be_2d)
        else:
            h, hwp = pl.pallas_call(
                _t2_body,
                grid=grid,
                in_specs=[_row_spec(D), _row_spec(D), accp_spec, dis_spec,
                          full(1, D), full(1, D), full(1, D), full(D, D)],
                out_specs=[_row_spec(D), _row_spec(D)],
                out_shape=[jax.ShapeDtypeStruct((N, D), jnp.float32),
                           jax.ShapeDtypeStruct((N, D), jnp.float32)],
            )(h, hwp, accp, dis, b_2d, g_2d, be_2d, Wn)
    return h


# speedup vs baseline: 1.1585x; 1.1585x over previous
import functools
import jax
import jax.numpy as jnp
from jax import lax
from jax.experimental import pallas as pl
from jax.experimental.pallas import tpu as pltpu
from jax.experimental.pallas import tpu_sc as plsc

NC = 2
NS = 16
NW = NC * NS
CHUNK = 128
RB = 1000
RC = 80
CG = 16


def _mesh():
    return plsc.VectorSubcoreMesh(core_axis_name="c", subcore_axis_name="s")


@functools.lru_cache(maxsize=None)
def _make_deg_kernel(N, EC, K):
    DG = 10
    nrc = N // RC
    rc_base, rc_rem = nrc // NS, nrc % NS

    @functools.partial(
        pl.kernel,
        out_type=(jax.ShapeDtypeStruct((N,), jnp.float32),
                  jax.ShapeDtypeStruct((N,), jnp.float32)),
        mesh=_mesh(),
        scratch_types=[
            pltpu.VMEM((DG, 2, CHUNK), jnp.int32),
            pltpu.VMEM((CHUNK,), jnp.float32),
            pltpu.VMEM((RC,), jnp.float32),
            pltpu.VMEM_SHARED((N,), jnp.float32),
            pltpu.SemaphoreType.DMA,
            pltpu.SemaphoreType.DMA,
        ],
    )
    def deg_kernel(eidx_hbm, ones_hbm, zeros_hbm, out0_hbm, out1_hbm,
                   idx_v, ones_v, buf_v, acc_sh, sa, sb):
        c = lax.axis_index("c")
        s = lax.axis_index("s")
        w = s * NC + c
        pltpu.sync_copy(zeros_hbm, buf_v)
        pltpu.sync_copy(ones_hbm, ones_v)
        nrc_mine = rc_base + jnp.where(s < rc_rem, 1, 0)

        def zbody(k, carry):
            pltpu.sync_copy(buf_v, acc_sh.at[pl.ds((s + NS * k) * RC, RC)])
            return carry

        lax.fori_loop(0, nrc_mine, zbody, 0)
        plsc.subcore_barrier()
        rw = jnp.clip(EC - w * K, 0, K)
        ng = rw // DG
        ssem = [sa, sb]

        def scat(j):
            return pltpu.async_copy(ones_v, acc_sh.at[idx_v.at[j, 1]],
                                    ssem[j % 2], add=True)

        def group(g, carry):
            pltpu.sync_copy(eidx_hbm.at[w, pl.ds(g * DG, DG)], idx_v)
            sd = [None] * DG
            for k in range(DG):
                if k >= 2:
                    sd[k - 2].wait()
                sd[k] = scat(k)
            sd[DG - 2].wait()
            sd[DG - 1].wait()
            return carry

        lax.fori_loop(0, ng, group, 0)
        plsc.subcore_barrier()

        def obody(k, carry):
            t = (s + NS * k) * RC
            pltpu.sync_copy(acc_sh.at[pl.ds(t, RC)], buf_v)

            @pl.when(c == 0)
            def _():
                pltpu.sync_copy(buf_v, out0_hbm.at[pl.ds(t, RC)])

            @pl.when(c == 1)
            def _():
                pltpu.sync_copy(buf_v, out1_hbm.at[pl.ds(t, RC)])

            return carry

        lax.fori_loop(0, nrc_mine, obody, 0)

    return deg_kernel


@functools.lru_cache(maxsize=None)
def _make_edge_kernel(N, D, K):
    nrc = N // RC
    rc_base, rc_rem = nrc // NS, nrc % NS
    NG = K // CG
    G2 = CG // 2

    @functools.partial(
        pl.kernel,
        out_type=jax.ShapeDtypeStruct((NC, N, D), jnp.float32),
        mesh=_mesh(),
        scratch_types=[
            pltpu.VMEM((CG, 2, CHUNK), jnp.int32),
            pltpu.VMEM((CHUNK, D), jnp.float32),
            pltpu.VMEM((CHUNK, D), jnp.float32),
            pltpu.VMEM_SHARED((N, D), jnp.float32),
            pltpu.SemaphoreType.DMA,
            pltpu.SemaphoreType.DMA,
        ],
    )
    def edge_kernel(hwp_hbm, eidx_hbm, zeros_hbm, out_hbm,
                    idx_v, rows_a, rows_b, acc_sh, ga, sa):
        c = lax.axis_index("c")
        s = lax.axis_index("s")
        w = s * NC + c
        pltpu.sync_copy(zeros_hbm, rows_a.at[pl.ds(0, RC)])
        nrc_mine = rc_base + jnp.where(s < rc_rem, 1, 0)

        def zbody(k, carry):
            pltpu.sync_copy(rows_a.at[pl.ds(0, RC)],
                            acc_sh.at[pl.ds((s + NS * k) * RC, RC)])
            return carry

        lax.fori_loop(0, nrc_mine, zbody, 0)
        plsc.subcore_barrier()

        def gather(j, rows, sem):
            return pltpu.async_copy(hwp_hbm.at[idx_v.at[j, 0]], rows, sem)

        def scat(j, rows, sem):
            return pltpu.async_copy(rows, acc_sh.at[idx_v.at[j, 1]], sem,
                                    add=True)

        rows = [rows_a, rows_b]

        def group(g, carry):
            pltpu.sync_copy(eidx_hbm.at[w, pl.ds(g * CG, CG)], idx_v)
            gd = [None] * CG
            sd = [None] * CG
            gd[0] = gather(0, rows[0], ga)
            for k in range(CG):
                gd[k].wait()
                if k > 0:
                    sd[k - 1].wait()
                sd[k] = scat(k, rows[k % 2], sa)
                if k + 1 < CG:
                    gd[k + 1] = gather(k + 1, rows[(k + 1) % 2], ga)
            sd[CG - 1].wait()
            return carry

        lax.fori_loop(0, NG, group, 0)
        plsc.subcore_barrier()

        def obody(k, carry):
            t = (s + NS * k) * RC
            pltpu.sync_copy(acc_sh.at[pl.ds(t, RC)], rows_a.at[pl.ds(0, RC)])
            pltpu.sync_copy(rows_a.at[pl.ds(0, RC)],
                            out_hbm.at[c, pl.ds(t, RC)])
            return carry

        lax.fori_loop(0, nrc_mine, obody, 0)

    return edge_kernel


def _t1_body(h_ref, w_ref, deg0_ref, deg1_ref, hwp_ref, dis_ref):
    d = deg0_ref[...] + deg1_ref[...] + 1.0
    dis = lax.rsqrt(d)
    dis_ref[...] = dis
    hwp_ref[...] = (
        jnp.dot(h_ref[...], w_ref[...], preferred_element_type=jnp.float32)
        * dis
    )


def _post(h, hwp, accp0, accp1, dis, b, g, be):
    out = (accp0 + accp1 + hwp) * dis + b
    mu = jnp.mean(out, axis=-1, keepdims=True)
    xm = out - mu
    var = jnp.mean(xm * xm, axis=-1, keepdims=True)
    out = xm * lax.rsqrt(var + 1e-5) * g + be
    return h + jnp.maximum(out, 0.0)


def _t2_body(h_ref, hwp_ref, accp_ref, dis_ref, b_ref, g_ref, be_ref, wn_ref,
             hn_ref, hwpn_ref):
    hn = _post(h_ref[...], hwp_ref[...], accp_ref[0], accp_ref[1],
               dis_ref[...], b_ref[...], g_ref[...], be_ref[...])
    hn_ref[...] = hn
    hwpn_ref[...] = (
        jnp.dot(hn, wn_ref[...], preferred_element_type=jnp.float32)
        * dis_ref[...]
    )


def _t3_body(h_ref, hwp_ref, accp_ref, dis_ref, b_ref, g_ref, be_ref, hn_ref):
    hn_ref[...] = _post(h_ref[...], hwp_ref[...], accp_ref[0], accp_ref[1],
                        dis_ref[...], b_ref[...], g_ref[...], be_ref[...])


def _row_spec(D):
    return pl.BlockSpec((RB, D), lambda i: (i, 0))


def kernel(x, edge_index, W0, b0, g0, be0, W1, b1, g1, be1, W2, b2, g2, be2):
    N, D = x.shape
    E = edge_index.shape[1]
    EC = E // CHUNK
    K = -(-EC // NW)
    K = -(-K // CG) * CG
    ECp = K * NW
    pad = ECp * CHUNK - E
    pad_src = N + (jnp.arange(pad, dtype=jnp.int32) % 8)
    pad_dst = jnp.arange(pad, dtype=jnp.int32) % N
    src_p = jnp.concatenate([edge_index[0], pad_src])
    dst_p = jnp.concatenate([edge_index[1], pad_dst])
    eidx = jnp.stack([src_p.reshape(ECp, CHUNK), dst_p.reshape(ECp, CHUNK)],
                     axis=1).reshape(NW, K, 2, CHUNK)
    zrow8 = jnp.zeros((8, D), jnp.float32)
    ones1 = jnp.ones((CHUNK,), jnp.float32)
    zdeg = jnp.zeros((RC,), jnp.float32)
    zrow = jnp.zeros((RC, D), jnp.float32)
    grid = (N // RB,)

    deg0, deg1 = _make_deg_kernel(N, EC, K)(eidx, ones1, zdeg)
    deg0 = deg0.reshape(N, 1)
    deg1 = deg1.reshape(N, 1)

    full = lambda a, b: pl.BlockSpec((a, b), lambda i: (0, 0))
    dis_spec = pl.BlockSpec((RB, 1), lambda i: (i, 0))
    accp_spec = pl.BlockSpec((NC, RB, D), lambda i: (0, i, 0))

    hwp, dis = pl.pallas_call(
        _t1_body,
        grid=grid,
        in_specs=[_row_spec(D), full(D, D), dis_spec, dis_spec],
        out_specs=[_row_spec(D), dis_spec],
        out_shape=[jax.ShapeDtypeStruct((N, D), jnp.float32),
                   jax.ShapeDtypeStruct((N, 1), jnp.float32)],
    )(x, W0, deg0, deg1)

    edge_k = _make_edge_kernel(N, D, K)
    h = x
    layer_params = [(b0, g0, be0, W1), (b1, g1, be1, W2), (b2, g2, be2, None)]
    for (b, g, be, Wn) in layer_params:
        accp = edge_k(jnp.concatenate([hwp, zrow8]), eidx, zrow)
        b_2d, g_2d, be_2d = b.reshape(1, D), g.reshape(1, D), be.reshape(1, D)
        if Wn is None:
            h = pl.pallas_call(
                _t3_body,
                grid=grid,
                in_specs=[_row_spec(D), _row_spec(D), accp_spec, dis_spec,
                          full(1, D), full(1, D), full(1, D)],
                out_specs=_row_spec(D),
                out_shape=jax.ShapeDtypeStruct((N, D), jnp.float32),
            )(h, hwp, accp, dis, b_2d, g_2d, be_2d)
        else:
            h, hwp = pl.pallas_call(
                _t2_body,
                grid=grid,
                in_specs=[_row_spec(D), _row_spec(D), accp_spec, dis_spec,
                          full(1, D), full(1, D), full(1, D), full(D, D)],
                out_specs=[_row_spec(D), _row_spec(D)],
                out_shape=[jax.ShapeDtypeStruct((N, D), jnp.float32),
                           jax.ShapeDtypeStruct((N, D), jnp.float32)],
            )(h, hwp, accp, dis, b_2d, g_2d, be_2d, Wn)
    return h
